# Initial kernel scaffold; baseline (speedup 1.0000x reference)
import jax
import jax.numpy as jnp
from jax.experimental import pallas as pl


def kernel(x, edge_index, W1, b1, W2, b2, W3, b3, W4, b4, W5, b5):
    raise NotImplementedError("write your pallas kernel here")


# speedup vs baseline: 18.5482x; 18.5482x over previous
import functools

import jax
import jax.numpy as jnp
from jax import lax
from jax.experimental import pallas as pl
from jax.experimental.pallas import tpu as pltpu
from jax.experimental.pallas import tpu_sc as plsc

N = 10000
NPAD = 10240
E_RAW = 320000
E_TOT = E_RAW + N
NTILES = 32
CHUNK = 128
NCHUNK = 81
EPAD = NTILES * NCHUNK * CHUNK
RPT = NPAD // 16

_MESH = plsc.VectorSubcoreMesh(core_axis_name="c", subcore_axis_name="s")


def _sc_degree(dst_r):
    @functools.partial(
        pl.kernel,
        out_type=jax.ShapeDtypeStruct((2, NPAD, 16), jnp.float32),
        mesh=_MESH,
        compiler_params=pltpu.CompilerParams(use_tc_tiling_on_sc=False),
        scratch_types=[
            pltpu.VMEM((NCHUNK, CHUNK), jnp.int32),
            pltpu.VMEM((CHUNK, 16), jnp.float32),
            pltpu.VMEM_SHARED((NPAD, 16), jnp.float32),
        ],
    )
    def k(dst_hbm, out_hbm, idx_v, ones_v, acc):
        c = lax.axis_index("c")
        s = lax.axis_index("s")
        wid = c * 16 + s
        pltpu.sync_copy(dst_hbm.at[wid], idx_v)

        def fill(i, val):
            def body(j, _):
                ones_v[j, :] = jnp.full((16,), val, jnp.float32)
                return 0
            return lax.fori_loop(0, CHUNK, body, 0)

        fill(0, 0.0)
        for t in range(RPT // CHUNK):
            pltpu.sync_copy(ones_v, acc.at[pl.ds(s * RPT + t * CHUNK, CHUNK)])
        fill(0, 1.0)
        plsc.subcore_barrier()

        def body(j, _):
            pltpu.sync_copy(ones_v, acc.at[idx_v.at[j]], add=True)
            return 0

        lax.fori_loop(0, NCHUNK, body, 0)
        plsc.subcore_barrier()
        pltpu.sync_copy(acc.at[pl.ds(s * RPT, RPT)],
                        out_hbm.at[c, pl.ds(s * RPT, RPT)])

    return k(dst_r)


def _sc_propagate(g, src_r, dst_r, width):
    @functools.partial(
        pl.kernel,
        out_type=jax.ShapeDtypeStruct((2, NPAD, width), jnp.float32),
        mesh=_MESH,
        compiler_params=pltpu.CompilerParams(use_tc_tiling_on_sc=False),
        scratch_types=[
            pltpu.VMEM((NCHUNK, CHUNK), jnp.int32),
            pltpu.VMEM((NCHUNK, CHUNK), jnp.int32),
            pltpu.VMEM((CHUNK, width), jnp.float32),
            pltpu.VMEM_SHARED((NPAD, width), jnp.float32),
            pltpu.SemaphoreType.DMA,
        ],
    )
    def k(g_hbm, src_hbm, dst_hbm, out_hbm, src_v, dst_v, rows_v, acc, sem):
        c = lax.axis_index("c")
        s = lax.axis_index("s")
        wid = c * 16 + s
        pltpu.sync_copy(src_hbm.at[wid], src_v)
        pltpu.sync_copy(dst_hbm.at[wid], dst_v)

        zeros16 = jnp.zeros((16,), jnp.float32)

        def zbody(i, _):
            for kk in range(width // 16):
                rows_v[i, pl.ds(kk * 16, 16)] = zeros16
            return 0

        lax.fori_loop(0, CHUNK, zbody, 0)
        for t in range(RPT // CHUNK):
            pltpu.sync_copy(rows_v, acc.at[pl.ds(s * RPT + t * CHUNK, CHUNK)])
        plsc.subcore_barrier()

        def body(j, _):
            pltpu.async_copy(g_hbm.at[src_v.at[j]], rows_v, sem).wait()
            pltpu.sync_copy(rows_v, acc.at[dst_v.at[j]], add=True)
            return 0

        lax.fori_loop(0, NCHUNK, body, 0)
        plsc.subcore_barrier()
        pltpu.sync_copy(acc.at[pl.ds(s * RPT, RPT)],
                        out_hbm.at[c, pl.ds(s * RPT, RPT)])

    return k(g, src_r, dst_r)


def _k0_body(dp_ref, x_ref, w_ref, g_ref, dis_ref):
    deg = (dp_ref[0] + dp_ref[1])[:, 0:1]
    row = lax.broadcasted_iota(jnp.int32, (NPAD, 1), 0)
    dis = jnp.where((deg > 0) & (row < N), lax.rsqrt(deg), 0.0)
    g_ref[...] = jnp.dot(x_ref[...], w_ref[...],
                         preferred_element_type=jnp.float32) * dis
    dis_ref[...] = dis


def _kmid_body(p_ref, dis_ref, b_ref, w_ref, g_ref):
    dis = dis_ref[...]
    z = jnp.maximum((p_ref[0] + p_ref[1]) * dis + b_ref[...], 0.0)
    g_ref[...] = jnp.dot(z, w_ref[...], preferred_element_type=jnp.float32) * dis


def _k4_body(p_ref, dis_ref, b_ref, g_ref):
    dis = dis_ref[...]
    g_ref[...] = jnp.maximum((p_ref[0] + p_ref[1]) * dis + b_ref[...], 0.0) * dis


def _k5_body(p_ref, dis_ref, b_ref, w_ref, o_ref):
    o_ref[...] = jnp.dot((p_ref[0] + p_ref[1]) * dis_ref[...], w_ref[...],
                         preferred_element_type=jnp.float32) + b_ref[...]


def kernel(x, edge_index, W1, b1, W2, b2, W3, b3, W4, b4, W5, b5):
    f32 = jnp.float32
    xp = jnp.concatenate([x, jnp.zeros((NPAD - N, x.shape[1]), f32)], axis=0)
    loop = jnp.arange(N, dtype=jnp.int32)
    padv = jnp.full((EPAD - E_TOT,), N, dtype=jnp.int32)
    src_r = jnp.concatenate(
        [edge_index[0].astype(jnp.int32), loop, padv]).reshape(NTILES, NCHUNK, CHUNK)
    dst_r = jnp.concatenate(
        [edge_index[1].astype(jnp.int32), loop, padv]).reshape(NTILES, NCHUNK, CHUNK)

    deg_parts = _sc_degree(dst_r)
    g, dis = pl.pallas_call(_k0_body, out_shape=(
        jax.ShapeDtypeStruct((NPAD, 128), f32),
        jax.ShapeDtypeStruct((NPAD, 1), f32)))(deg_parts, xp, W1)

    p = _sc_propagate(g, src_r, dst_r, 128)
    g = pl.pallas_call(_kmid_body, out_shape=jax.ShapeDtypeStruct(
        (NPAD, 64), f32))(p, dis, b1.reshape(1, -1), W2)
    p = _sc_propagate(g, src_r, dst_r, 64)
    g = pl.pallas_call(_kmid_body, out_shape=jax.ShapeDtypeStruct(
        (NPAD, 32), f32))(p, dis, b2.reshape(1, -1), W3)
    p = _sc_propagate(g, src_r, dst_r, 32)
    g = pl.pallas_call(_kmid_body, out_shape=jax.ShapeDtypeStruct(
        (NPAD, 16), f32))(p, dis, b3.reshape(1, -1), W4)
    p = _sc_propagate(g, src_r, dst_r, 16)
    g = pl.pallas_call(_k4_body, out_shape=jax.ShapeDtypeStruct(
        (NPAD, 16), f32))(p, dis, b4.reshape(1, -1))
    p = _sc_propagate(g, src_r, dst_r, 16)
    out = pl.pallas_call(_k5_body, out_shape=jax.ShapeDtypeStruct(
        (NPAD, 40), f32))(p, dis, b5.reshape(1, -1), W5)
    return out[:N]


# speedup vs baseline: 22.9265x; 1.2361x over previous
import functools

import jax
import jax.numpy as jnp
from jax import lax
from jax.experimental import pallas as pl
from jax.experimental.pallas import tpu as pltpu
from jax.experimental.pallas import tpu_sc as plsc

N = 10000
NPAD = 10240
E_RAW = 320000
E_TOT = E_RAW + N
NTILES = 32
CHUNK = 128
NCHUNK = 81
EPAD = NTILES * NCHUNK * CHUNK
RPT = NPAD // 16

_MESH = plsc.VectorSubcoreMesh(core_axis_name="c", subcore_axis_name="s")


def _sc_degree(dst_r):
    @functools.partial(
        pl.kernel,
        out_type=jax.ShapeDtypeStruct((2, NPAD, 16), jnp.float32),
        mesh=_MESH,
        compiler_params=pltpu.CompilerParams(use_tc_tiling_on_sc=False),
        scratch_types=[
            pltpu.VMEM((NCHUNK, CHUNK), jnp.int32),
            pltpu.VMEM((CHUNK, 16), jnp.float32),
            pltpu.VMEM_SHARED((NPAD, 16), jnp.float32),
        ],
    )
    def k(dst_hbm, out_hbm, idx_v, ones_v, acc):
        c = lax.axis_index("c")
        s = lax.axis_index("s")
        wid = c * 16 + s
        pltpu.sync_copy(dst_hbm.at[wid], idx_v)

        def fill(i, val):
            def body(j, _):
                ones_v[j, :] = jnp.full((16,), val, jnp.float32)
                return 0
            return lax.fori_loop(0, CHUNK, body, 0)

        fill(0, 0.0)
        for t in range(RPT // CHUNK):
            pltpu.sync_copy(ones_v, acc.at[pl.ds(s * RPT + t * CHUNK, CHUNK)])
        fill(0, 1.0)
        plsc.subcore_barrier()

        def body(j, _):
            pltpu.sync_copy(ones_v, acc.at[idx_v.at[j]], add=True)
            return 0

        lax.fori_loop(0, NCHUNK, body, 0)
        plsc.subcore_barrier()
        pltpu.sync_copy(acc.at[pl.ds(s * RPT, RPT)],
                        out_hbm.at[c, pl.ds(s * RPT, RPT)])

    return k(dst_r)


def _sc_propagate(g, src_r, dst_r, width):
    @functools.partial(
        pl.kernel,
        out_type=jax.ShapeDtypeStruct((2, NPAD, width), jnp.float32),
        mesh=_MESH,
        compiler_params=pltpu.CompilerParams(use_tc_tiling_on_sc=False),
        scratch_types=[
            pltpu.VMEM((NCHUNK, CHUNK), jnp.int32),
            pltpu.VMEM((NCHUNK, CHUNK), jnp.int32),
            pltpu.VMEM((2 if width <= 64 else 1, CHUNK, width), jnp.float32),
            pltpu.VMEM_SHARED((NPAD, width), jnp.float32),
            pltpu.SemaphoreType.DMA((2,)),
            pltpu.SemaphoreType.DMA,
        ],
    )
    def k(g_hbm, src_hbm, dst_hbm, out_hbm, src_v, dst_v, rows_v, acc, gsem, isem):
        c = lax.axis_index("c")
        s = lax.axis_index("s")
        wid = c * 16 + s
        pltpu.async_copy(src_hbm.at[wid], src_v, isem)
        pltpu.async_copy(dst_hbm.at[wid], dst_v, isem)

        zeros16 = jnp.zeros((16,), jnp.float32)

        def zbody(i, _):
            for kk in range(width // 16):
                rows_v[0, i, pl.ds(kk * 16, 16)] = zeros16
            return 0

        lax.fori_loop(0, CHUNK, zbody, 0)
        for t in range(RPT // CHUNK):
            pltpu.sync_copy(rows_v.at[0], acc.at[pl.ds(s * RPT + t * CHUNK, CHUNK)])
        plsc.subcore_barrier()
        pltpu.make_async_copy(src_hbm.at[wid], src_v, isem).wait()
        pltpu.make_async_copy(dst_hbm.at[wid], dst_v, isem).wait()

        if width <= 64:
            pltpu.async_copy(g_hbm.at[src_v.at[0]], rows_v.at[0], gsem.at[0])

            def body(j, _):
                par = lax.rem(j, 2)
                npar = lax.rem(j + 1, 2)

                @pl.when(j < NCHUNK - 1)
                def _():
                    pltpu.async_copy(g_hbm.at[src_v.at[j + 1]], rows_v.at[npar],
                                     gsem.at[npar])

                pltpu.make_async_copy(g_hbm.at[src_v.at[j]], rows_v.at[par],
                                      gsem.at[par]).wait()
                pltpu.sync_copy(rows_v.at[par], acc.at[dst_v.at[j]], add=True)
                return 0
        else:
            def body(j, _):
                pltpu.async_copy(g_hbm.at[src_v.at[j]], rows_v.at[0],
                                 gsem.at[0]).wait()
                pltpu.sync_copy(rows_v.at[0], acc.at[dst_v.at[j]], add=True)
                return 0

        lax.fori_loop(0, NCHUNK, body, 0)
        plsc.subcore_barrier()
        pltpu.sync_copy(acc.at[pl.ds(s * RPT, RPT)],
                        out_hbm.at[c, pl.ds(s * RPT, RPT)])

    return k(g, src_r, dst_r)


def _k0_body(dp_ref, x_ref, w_ref, g_ref, dis_ref):
    deg = (dp_ref[0] + dp_ref[1])[:, 0:1]
    row = lax.broadcasted_iota(jnp.int32, (NPAD, 1), 0)
    dis = jnp.where((deg > 0) & (row < N), lax.rsqrt(deg), 0.0)
    g_ref[...] = jnp.dot(x_ref[...], w_ref[...],
                         preferred_element_type=jnp.float32) * dis
    dis_ref[...] = dis


def _kmid_body(p_ref, dis_ref, b_ref, w_ref, g_ref):
    dis = dis_ref[...]
    z = jnp.maximum((p_ref[0] + p_ref[1]) * dis + b_ref[...], 0.0)
    g_ref[...] = jnp.dot(z, w_ref[...], preferred_element_type=jnp.float32) * dis


def _k4_body(p_ref, dis_ref, b_ref, g_ref):
    dis = dis_ref[...]
    g_ref[...] = jnp.maximum((p_ref[0] + p_ref[1]) * dis + b_ref[...], 0.0) * dis


def _k5_body(p_ref, dis_ref, b_ref, w_ref, o_ref):
    o_ref[...] = jnp.dot((p_ref[0] + p_ref[1]) * dis_ref[...], w_ref[...],
                         preferred_element_type=jnp.float32) + b_ref[...]


def kernel(x, edge_index, W1, b1, W2, b2, W3, b3, W4, b4, W5, b5):
    f32 = jnp.float32
    xp = jnp.concatenate([x, jnp.zeros((NPAD - N, x.shape[1]), f32)], axis=0)
    loop = jnp.arange(N, dtype=jnp.int32)
    padv = jnp.full((EPAD - E_TOT,), N, dtype=jnp.int32)
    src_r = jnp.concatenate(
        [edge_index[0].astype(jnp.int32), loop, padv]).reshape(NTILES, NCHUNK, CHUNK)
    dst_r = jnp.concatenate(
        [edge_index[1].astype(jnp.int32), loop, padv]).reshape(NTILES, NCHUNK, CHUNK)

    deg_parts = _sc_degree(dst_r)
    g, dis = pl.pallas_call(_k0_body, out_shape=(
        jax.ShapeDtypeStruct((NPAD, 128), f32),
        jax.ShapeDtypeStruct((NPAD, 1), f32)))(deg_parts, xp, W1)

    p = _sc_propagate(g, src_r, dst_r, 128)
    g = pl.pallas_call(_kmid_body, out_shape=jax.ShapeDtypeStruct(
        (NPAD, 64), f32))(p, dis, b1.reshape(1, -1), W2)
    p = _sc_propagate(g, src_r, dst_r, 64)
    g = pl.pallas_call(_kmid_body, out_shape=jax.ShapeDtypeStruct(
        (NPAD, 32), f32))(p, dis, b2.reshape(1, -1), W3)
    p = _sc_propagate(g, src_r, dst_r, 32)
    g = pl.pallas_call(_kmid_body, out_shape=jax.ShapeDtypeStruct(
        (NPAD, 16), f32))(p, dis, b3.reshape(1, -1), W4)
    p = _sc_propagate(g, src_r, dst_r, 16)
    g = pl.pallas_call(_k4_body, out_shape=jax.ShapeDtypeStruct(
        (NPAD, 16), f32))(p, dis, b4.reshape(1, -1))
    p = _sc_propagate(g, src_r, dst_r, 16)
    out = pl.pallas_call(_k5_body, out_shape=jax.ShapeDtypeStruct(
        (NPAD, 40), f32))(p, dis, b5.reshape(1, -1), W5)
    return out[:N]


# speedup vs baseline: 24.6262x; 1.0741x over previous
import functools

import jax
import jax.numpy as jnp
from jax import lax
from jax.experimental import pallas as pl
from jax.experimental.pallas import tpu as pltpu
from jax.experimental.pallas import tpu_sc as plsc

N = 10000
NPAD = 10240
E_RAW = 320000
E_TOT = E_RAW + N
NTILES = 32
CHUNK = 128
NCHUNK = 81
EPAD = NTILES * NCHUNK * CHUNK
RPT = NPAD // 16

_MESH = plsc.VectorSubcoreMesh(core_axis_name="c", subcore_axis_name="s")


def _sc_degree(dst_r):
    @functools.partial(
        pl.kernel,
        out_type=jax.ShapeDtypeStruct((2, NPAD, 16), jnp.float32),
        mesh=_MESH,
        compiler_params=pltpu.CompilerParams(use_tc_tiling_on_sc=False),
        scratch_types=[
            pltpu.VMEM((NCHUNK, CHUNK), jnp.int32),
            pltpu.VMEM((CHUNK, 16), jnp.float32),
            pltpu.VMEM_SHARED((NPAD, 16), jnp.float32),
        ],
    )
    def k(dst_hbm, out_hbm, idx_v, ones_v, acc):
        c = lax.axis_index("c")
        s = lax.axis_index("s")
        wid = c * 16 + s
        pltpu.sync_copy(dst_hbm.at[wid], idx_v)

        def fill(i, val):
            def body(j, _):
                ones_v[j, :] = jnp.full((16,), val, jnp.float32)
                return 0
            return lax.fori_loop(0, CHUNK, body, 0)

        fill(0, 0.0)
        for t in range(RPT // CHUNK):
            pltpu.sync_copy(ones_v, acc.at[pl.ds(s * RPT + t * CHUNK, CHUNK)])
        fill(0, 1.0)
        plsc.subcore_barrier()

        def body(j, _):
            pltpu.sync_copy(ones_v, acc.at[idx_v.at[j]], add=True)
            return 0

        lax.fori_loop(0, NCHUNK, body, 0)
        plsc.subcore_barrier()
        pltpu.sync_copy(acc.at[pl.ds(s * RPT, RPT)],
                        out_hbm.at[c, pl.ds(s * RPT, RPT)])

    return k(dst_r)


def _sc_propagate(g, src_r, dst_r, width):
    chunk = 64 if width >= 128 else CHUNK
    nchunk = EPAD // NTILES // chunk

    @functools.partial(
        pl.kernel,
        out_type=jax.ShapeDtypeStruct((2, NPAD, width), jnp.float32),
        mesh=_MESH,
        compiler_params=pltpu.CompilerParams(use_tc_tiling_on_sc=False),
        scratch_types=[
            pltpu.VMEM((nchunk, chunk), jnp.int32),
            pltpu.VMEM((nchunk, chunk), jnp.int32),
            pltpu.VMEM((2, chunk, width), jnp.float32),
            pltpu.VMEM_SHARED((NPAD, width), jnp.float32),
            pltpu.SemaphoreType.DMA((2,)),
            pltpu.SemaphoreType.DMA,
        ],
    )
    def k(g_hbm, src_hbm, dst_hbm, out_hbm, src_v, dst_v, rows_v, acc, gsem, isem):
        c = lax.axis_index("c")
        s = lax.axis_index("s")
        wid = c * 16 + s
        pltpu.async_copy(src_hbm.at[wid], src_v, isem)
        pltpu.async_copy(dst_hbm.at[wid], dst_v, isem)

        zeros16 = jnp.zeros((16,), jnp.float32)

        def zbody(i, _):
            for kk in range(width // 16):
                rows_v[0, i, pl.ds(kk * 16, 16)] = zeros16
            return 0

        lax.fori_loop(0, chunk, zbody, 0)
        for t in range(RPT // chunk):
            pltpu.sync_copy(rows_v.at[0], acc.at[pl.ds(s * RPT + t * chunk, chunk)])
        plsc.subcore_barrier()
        pltpu.make_async_copy(src_hbm.at[wid], src_v, isem).wait()
        pltpu.make_async_copy(dst_hbm.at[wid], dst_v, isem).wait()

        pltpu.async_copy(g_hbm.at[src_v.at[0]], rows_v.at[0], gsem.at[0])

        def body(j, _):
            par = lax.rem(j, 2)
            npar = lax.rem(j + 1, 2)

            @pl.when(j < nchunk - 1)
            def _():
                pltpu.async_copy(g_hbm.at[src_v.at[j + 1]], rows_v.at[npar],
                                 gsem.at[npar])

            pltpu.make_async_copy(g_hbm.at[src_v.at[j]], rows_v.at[par],
                                  gsem.at[par]).wait()
            pltpu.sync_copy(rows_v.at[par], acc.at[dst_v.at[j]], add=True)
            return 0

        lax.fori_loop(0, nchunk, body, 0)
        plsc.subcore_barrier()
        pltpu.sync_copy(acc.at[pl.ds(s * RPT, RPT)],
                        out_hbm.at[c, pl.ds(s * RPT, RPT)])

    return k(g, src_r.reshape(NTILES, nchunk, chunk),
             dst_r.reshape(NTILES, nchunk, chunk))


def _k0_body(dp_ref, x_ref, w_ref, g_ref, dis_ref):
    deg = (dp_ref[0] + dp_ref[1])[:, 0:1]
    row = lax.broadcasted_iota(jnp.int32, (NPAD, 1), 0)
    dis = jnp.where((deg > 0) & (row < N), lax.rsqrt(deg), 0.0)
    g_ref[...] = jnp.dot(x_ref[...], w_ref[...],
                         preferred_element_type=jnp.float32) * dis
    dis_ref[...] = dis


def _kmid_body(p_ref, dis_ref, b_ref, w_ref, g_ref):
    dis = dis_ref[...]
    z = jnp.maximum((p_ref[0] + p_ref[1]) * dis + b_ref[...], 0.0)
    g_ref[...] = jnp.dot(z, w_ref[...], preferred_element_type=jnp.float32) * dis


def _k4_body(p_ref, dis_ref, b_ref, g_ref):
    dis = dis_ref[...]
    g_ref[...] = jnp.maximum((p_ref[0] + p_ref[1]) * dis + b_ref[...], 0.0) * dis


def _k5_body(p_ref, dis_ref, b_ref, w_ref, o_ref):
    o_ref[...] = jnp.dot((p_ref[0] + p_ref[1]) * dis_ref[...], w_ref[...],
                         preferred_element_type=jnp.float32) + b_ref[...]


def kernel(x, edge_index, W1, b1, W2, b2, W3, b3, W4, b4, W5, b5):
    f32 = jnp.float32
    xp = jnp.concatenate([x, jnp.zeros((NPAD - N, x.shape[1]), f32)], axis=0)
    loop = jnp.arange(N, dtype=jnp.int32)
    padv = jnp.full((EPAD - E_TOT,), N, dtype=jnp.int32)
    src_r = jnp.concatenate(
        [edge_index[0].astype(jnp.int32), loop, padv]).reshape(NTILES, NCHUNK, CHUNK)
    dst_r = jnp.concatenate(
        [edge_index[1].astype(jnp.int32), loop, padv]).reshape(NTILES, NCHUNK, CHUNK)

    deg_parts = _sc_degree(dst_r)
    g, dis = pl.pallas_call(_k0_body, out_shape=(
        jax.ShapeDtypeStruct((NPAD, 128), f32),
        jax.ShapeDtypeStruct((NPAD, 1), f32)))(deg_parts, xp, W1)

    p = _sc_propagate(g, src_r, dst_r, 128)
    g = pl.pallas_call(_kmid_body, out_shape=jax.ShapeDtypeStruct(
        (NPAD, 64), f32))(p, dis, b1.reshape(1, -1), W2)
    p = _sc_propagate(g, src_r, dst_r, 64)
    g = pl.pallas_call(_kmid_body, out_shape=jax.ShapeDtypeStruct(
        (NPAD, 32), f32))(p, dis, b2.reshape(1, -1), W3)
    p = _sc_propagate(g, src_r, dst_r, 32)
    g = pl.pallas_call(_kmid_body, out_shape=jax.ShapeDtypeStruct(
        (NPAD, 16), f32))(p, dis, b3.reshape(1, -1), W4)
    p = _sc_propagate(g, src_r, dst_r, 16)
    g = pl.pallas_call(_k4_body, out_shape=jax.ShapeDtypeStruct(
        (NPAD, 16), f32))(p, dis, b4.reshape(1, -1))
    p = _sc_propagate(g, src_r, dst_r, 16)
    out = pl.pallas_call(_k5_body, out_shape=jax.ShapeDtypeStruct(
        (NPAD, 40), f32))(p, dis, b5.reshape(1, -1), W5)
    return out[:N]


# speedup vs baseline: 26.7582x; 1.0866x over previous
import functools

import jax
import jax.numpy as jnp
from jax import lax
from jax.experimental import pallas as pl
from jax.experimental.pallas import tpu as pltpu
from jax.experimental.pallas import tpu_sc as plsc

N = 10000
NPAD = 10240
E_RAW = 320000
E_TOT = E_RAW + N
NTILES = 32
EPAD = 331776
PAD1D = EPAD + 12800
RPT = NPAD // 16

_MESH = plsc.VectorSubcoreMesh(core_axis_name="c", subcore_axis_name="s")

_SPLIT = {128: (64, 197), 64: (128, 100), 32: (128, 93), 16: (128, 86)}


def _sc_degree(dst_r):
    nmax = 100

    @functools.partial(
        pl.kernel,
        out_type=jax.ShapeDtypeStruct((2, NPAD, 16), jnp.float32),
        mesh=_MESH,
        compiler_params=pltpu.CompilerParams(use_tc_tiling_on_sc=False),
        scratch_types=[
            pltpu.VMEM((nmax, 128), jnp.int32),
            pltpu.VMEM((128, 16), jnp.float32),
            pltpu.VMEM_SHARED((NPAD, 16), jnp.float32),
            pltpu.SemaphoreType.DMA,
        ],
    )
    def k(dst_hbm, out_hbm, idx_v, ones_v, acc, isem):
        c = lax.axis_index("c")
        s = lax.axis_index("s")
        start = s * 162 + c * 81
        pltpu.async_copy(dst_hbm.at[pl.ds(start, nmax)], idx_v, isem)

        def fill(val):
            def body(j, _):
                ones_v[j, :] = jnp.full((16,), val, jnp.float32)
                return 0
            return lax.fori_loop(0, 128, body, 0)

        fill(0.0)
        for t in range(RPT // 128):
            pltpu.sync_copy(ones_v, acc.at[pl.ds(s * RPT + t * 128, 128)])
        fill(1.0)
        plsc.subcore_barrier()
        pltpu.make_async_copy(dst_hbm.at[pl.ds(start, nmax)], idx_v, isem).wait()

        def body(j, _):
            pltpu.sync_copy(ones_v, acc.at[idx_v.at[j]], add=True)
            return 0

        lax.fori_loop(0, 81, body, 0)
        plsc.subcore_barrier()
        pltpu.sync_copy(acc.at[pl.ds(s * RPT, RPT)],
                        out_hbm.at[c, pl.ds(s * RPT, RPT)])

    return k(dst_r)


def _sc_propagate(g, src_f, dst_f, width):
    chunk, n0 = _SPLIT[width]
    t16 = EPAD // chunk // 16
    n1 = t16 - n0
    nmax = max(n0, n1)

    @functools.partial(
        pl.kernel,
        out_type=jax.ShapeDtypeStruct((2, NPAD, width), jnp.float32),
        mesh=_MESH,
        compiler_params=pltpu.CompilerParams(use_tc_tiling_on_sc=False),
        scratch_types=[
            pltpu.VMEM((nmax, chunk), jnp.int32),
            pltpu.VMEM((nmax, chunk), jnp.int32),
            pltpu.VMEM((2, chunk, width), jnp.float32),
            pltpu.VMEM_SHARED((NPAD, width), jnp.float32),
            pltpu.SemaphoreType.DMA((2,)),
            pltpu.SemaphoreType.DMA,
        ],
    )
    def k(g_hbm, src_hbm, dst_hbm, out_hbm, src_v, dst_v, rows_v, acc, gsem, isem):
        c = lax.axis_index("c")
        s = lax.axis_index("s")
        start = s * t16 + c * n0
        cnt = jnp.where(c == 0, n0, n1)
        pltpu.async_copy(src_hbm.at[pl.ds(start, nmax)], src_v, isem)
        pltpu.async_copy(dst_hbm.at[pl.ds(start, nmax)], dst_v, isem)

        zeros16 = jnp.zeros((16,), jnp.float32)

        def zbody(i, _):
            for kk in range(width // 16):
                rows_v[0, i, pl.ds(kk * 16, 16)] = zeros16
            return 0

        lax.fori_loop(0, chunk, zbody, 0)
        for t in range(RPT // chunk):
            pltpu.sync_copy(rows_v.at[0], acc.at[pl.ds(s * RPT + t * chunk, chunk)])
        plsc.subcore_barrier()
        pltpu.make_async_copy(src_hbm.at[pl.ds(start, nmax)], src_v, isem).wait()
        pltpu.make_async_copy(dst_hbm.at[pl.ds(start, nmax)], dst_v, isem).wait()

        pltpu.async_copy(g_hbm.at[src_v.at[0]], rows_v.at[0], gsem.at[0])

        def body(j, _):
            par = lax.rem(j, 2)
            npar = lax.rem(j + 1, 2)

            @pl.when(j < cnt - 1)
            def _():
                pltpu.async_copy(g_hbm.at[src_v.at[j + 1]], rows_v.at[npar],
                                 gsem.at[npar])

            pltpu.make_async_copy(g_hbm.at[src_v.at[j]], rows_v.at[par],
                                  gsem.at[par]).wait()
            pltpu.sync_copy(rows_v.at[par], acc.at[dst_v.at[j]], add=True)
            return 0

        lax.fori_loop(0, cnt, body, 0)
        plsc.subcore_barrier()
        pltpu.sync_copy(acc.at[pl.ds(s * RPT, RPT)],
                        out_hbm.at[c, pl.ds(s * RPT, RPT)])

    return k(g, src_f, dst_f)


def _k0_body(dp_ref, x_ref, w_ref, g_ref, dis_ref):
    deg = (dp_ref[0] + dp_ref[1])[:, 0:1]
    row = lax.broadcasted_iota(jnp.int32, (NPAD, 1), 0)
    dis = jnp.where((deg > 0) & (row < N), lax.rsqrt(deg), 0.0)
    g_ref[...] = jnp.dot(x_ref[...], w_ref[...],
                         preferred_element_type=jnp.float32) * dis
    dis_ref[...] = dis


def _kmid_body(p_ref, dis_ref, b_ref, w_ref, g_ref):
    dis = dis_ref[...]
    z = jnp.maximum((p_ref[0] + p_ref[1]) * dis + b_ref[...], 0.0)
    g_ref[...] = jnp.dot(z, w_ref[...], preferred_element_type=jnp.float32) * dis


def _k4_body(p_ref, dis_ref, b_ref, g_ref):
    dis = dis_ref[...]
    g_ref[...] = jnp.maximum((p_ref[0] + p_ref[1]) * dis + b_ref[...], 0.0) * dis


def _k5_body(p_ref, dis_ref, b_ref, w_ref, o_ref):
    o_ref[...] = jnp.dot((p_ref[0] + p_ref[1]) * dis_ref[...], w_ref[...],
                         preferred_element_type=jnp.float32) + b_ref[...]


def _views(flat1d, chunk, n0):
    t16 = EPAD // chunk // 16
    nmax = max(n0, t16 - n0)
    rows = EPAD // chunk + nmax
    return lax.slice(flat1d, (0,), (rows * chunk,)).reshape(rows, chunk)


def kernel(x, edge_index, W1, b1, W2, b2, W3, b3, W4, b4, W5, b5):
    f32 = jnp.float32
    xp = jnp.concatenate([x, jnp.zeros((NPAD - N, x.shape[1]), f32)], axis=0)
    loop = jnp.arange(N, dtype=jnp.int32)
    padv = jnp.full((PAD1D - E_TOT,), N, dtype=jnp.int32)
    src1d = jnp.concatenate([edge_index[0].astype(jnp.int32), loop, padv])
    dst1d = jnp.concatenate([edge_index[1].astype(jnp.int32), loop, padv])
    src_w = {w: _views(src1d, c, n) for w, (c, n) in _SPLIT.items()}
    dst_w = {w: _views(dst1d, c, n) for w, (c, n) in _SPLIT.items()}

    deg_parts = _sc_degree(dst_w[64])
    g, dis = pl.pallas_call(_k0_body, out_shape=(
        jax.ShapeDtypeStruct((NPAD, 128), f32),
        jax.ShapeDtypeStruct((NPAD, 1), f32)))(deg_parts, xp, W1)

    p = _sc_propagate(g, src_w[128], dst_w[128], 128)
    g = pl.pallas_call(_kmid_body, out_shape=jax.ShapeDtypeStruct(
        (NPAD, 64), f32))(p, dis, b1.reshape(1, -1), W2)
    p = _sc_propagate(g, src_w[64], dst_w[64], 64)
    g = pl.pallas_call(_kmid_body, out_shape=jax.ShapeDtypeStruct(
        (NPAD, 32), f32))(p, dis, b2.reshape(1, -1), W3)
    p = _sc_propagate(g, src_w[32], dst_w[32], 32)
    g = pl.pallas_call(_kmid_body, out_shape=jax.ShapeDtypeStruct(
        (NPAD, 16), f32))(p, dis, b3.reshape(1, -1), W4)
    p = _sc_propagate(g, src_w[16], dst_w[16], 16)
    g = pl.pallas_call(_k4_body, out_shape=jax.ShapeDtypeStruct(
        (NPAD, 16), f32))(p, dis, b4.reshape(1, -1))
    p = _sc_propagate(g, src_w[16], dst_w[16], 16)
    out = pl.pallas_call(_k5_body, out_shape=jax.ShapeDtypeStruct(
        (NPAD, 40), f32))(p, dis, b5.reshape(1, -1), W5)
    return out[:N]
